# Initial kernel scaffold; baseline (speedup 1.0000x reference)
import functools

import jax
import jax.numpy as jnp
from jax import lax
from jax.experimental import pallas as pl
from jax.experimental.pallas import tpu as pltpu
from jax.experimental.pallas import tpu_sc as plsc

N = 10000
E = 320000
D = 128

NC = 2
NS = 16
NW = NC * NS
EPW = E // NW
CHUNK = 80
NCHUNK = EPW // CHUNK
RPS = N // NS


def _sc_aggregate_call(x, src, dst, zf, zc, ones_c):
    mesh = plsc.VectorSubcoreMesh(core_axis_name="c", subcore_axis_name="s")

    @functools.partial(
        pl.kernel,
        out_type=[
            jax.ShapeDtypeStruct((NC * N, D), jnp.float32),
            jax.ShapeDtypeStruct((NC * N, 16), jnp.float32),
        ],
        mesh=mesh,
        scratch_types=[
            pltpu.VMEM((CHUNK,), jnp.int32),
            pltpu.VMEM((CHUNK,), jnp.int32),
            pltpu.VMEM((CHUNK, D), jnp.float32),
            pltpu.VMEM((CHUNK, 16), jnp.float32),
            pltpu.VMEM_SHARED((N, D), jnp.float32),
            pltpu.VMEM_SHARED((N, 16), jnp.float32),
            pltpu.SemaphoreType.DMA,
        ],
    )
    def k(x_hbm, src_hbm, dst_hbm, zf_hbm, zc_hbm, ones_hbm,
          outf_hbm, outc_hbm, srcv, dstv, rows, onesv, acc, cacc, sem):
        c = lax.axis_index("c")
        s = lax.axis_index("s")
        wid = s * NC + c

        pltpu.sync_copy(zf_hbm.at[pl.ds(s * RPS, RPS)], acc.at[pl.ds(s * RPS, RPS)])
        pltpu.sync_copy(zc_hbm.at[pl.ds(s * RPS, RPS)], cacc.at[pl.ds(s * RPS, RPS)])
        pltpu.sync_copy(ones_hbm, onesv)
        plsc.subcore_barrier()

        base = wid * EPW

        def body(i, carry):
            b = base + i * CHUNK
            pltpu.sync_copy(src_hbm.at[pl.ds(b, CHUNK)], srcv)
            pltpu.sync_copy(dst_hbm.at[pl.ds(b, CHUNK)], dstv)
            pltpu.async_copy(x_hbm.at[srcv], rows, sem).wait()
            pltpu.sync_copy(rows, acc.at[dstv], add=True)
            pltpu.sync_copy(onesv, cacc.at[dstv], add=True)
            return carry

        lax.fori_loop(0, NCHUNK, body, 0)
        plsc.subcore_barrier()

        ob = c * N + s * RPS
        pltpu.sync_copy(acc.at[pl.ds(s * RPS, RPS)], outf_hbm.at[pl.ds(ob, RPS)])
        pltpu.sync_copy(cacc.at[pl.ds(s * RPS, RPS)], outc_hbm.at[pl.ds(ob, RPS)])

    return k(x, src, dst, zf, zc, ones_c)


def _elu(v):
    return jnp.where(v > 0, v, jnp.expm1(jnp.minimum(v, 0.0)))


def _softplus(v):
    return jnp.maximum(v, 0.0) + jnp.log1p(jnp.exp(-jnp.abs(v)))


def _dot_t(a, w):
    return lax.dot_general(a, w, (((1,), (1,)), ((), ())),
                           preferred_element_type=jnp.float32)


TCB = 1000


def _tc1_body(x, pf, pc, wl1, bl1, wr1, r1w, r1b, r2w, r2b,
              h1_o, res1_o, res2_o):
    agg = pf[0] + pf[1]
    cnt = (pc[0] + pc[1])[:, 0:1]
    mean = agg / jnp.maximum(cnt, 1.0)
    lin = _dot_t(mean, wl1[...]) + bl1[...] + _dot_t(x[...], wr1[...])
    h1_o[...] = _elu(lin)
    res1_o[...] = _dot_t(x[...], r1w[...]) + r1b[...]
    res2_o[...] = _dot_t(x[...], r2w[...]) + r2b[...]


def _tc2_body(h1, pf, pc, res1, res2, wl2, bl2, wr2,
              f1w, f1b, f2w, f2b, f3w, f3b, out_o):
    agg = pf[0] + pf[1]
    cnt = (pc[0] + pc[1])[:, 0:1]
    mean = agg / jnp.maximum(cnt, 1.0)
    h = _elu(_dot_t(mean, wl2[...]) + bl2[...] + _dot_t(h1[...], wr2[...]))
    h = h + res1[...]
    h = _elu(_dot_t(h, f1w[...]) + f1b[...])
    h = _elu(_dot_t(h, f2w[...]) + f2b[...]) + res2[...]
    out_o[...] = _softplus(_dot_t(h, f3w[...]) + f3b[...])


def _row_spec(width=D):
    return pl.BlockSpec((TCB, width), lambda i: (i, 0))


def _part_spec(width):
    return pl.BlockSpec((NC, TCB, width), lambda i: (0, i, 0))


def _w_spec():
    return pl.BlockSpec((D, D), lambda i: (0, 0))


def _b_spec():
    return pl.BlockSpec((1, D), lambda i: (0, 0))


def kernel(x, edge_index, conv1_Wl, conv1_bl, conv1_Wr, conv2_Wl, conv2_bl,
           conv2_Wr, res1_W, res1_b, res2_W, res2_b, fc1_W, fc1_b, fc2_W,
           fc2_b, fc3_W, fc3_b):
    src = edge_index[0].astype(jnp.int32)
    dst = edge_index[1].astype(jnp.int32)
    zf = jnp.zeros((N, D), jnp.float32)
    zc = jnp.zeros((N, 16), jnp.float32)
    ones_c = jnp.ones((CHUNK, 16), jnp.float32)

    pf1, pc1 = _sc_aggregate_call(x, src, dst, zf, zc, ones_c)
    pf1 = pf1.reshape(NC, N, D)
    pc1 = pc1.reshape(NC, N, 16)

    grid = (N // TCB,)
    h1, res1, res2 = pl.pallas_call(
        _tc1_body,
        grid=grid,
        in_specs=[
            _row_spec(), _part_spec(D), _part_spec(16),
            _w_spec(), _b_spec(), _w_spec(),
            _w_spec(), _b_spec(), _w_spec(), _b_spec(),
        ],
        out_specs=[_row_spec(), _row_spec(), _row_spec()],
        out_shape=[jax.ShapeDtypeStruct((N, D), jnp.float32)] * 3,
    )(x, pf1, pc1, conv1_Wl, conv1_bl.reshape(1, D), conv1_Wr,
      res1_W, res1_b.reshape(1, D), res2_W, res2_b.reshape(1, D))

    pf2, pc2 = _sc_aggregate_call(h1, src, dst, zf, zc, ones_c)
    pf2 = pf2.reshape(NC, N, D)
    pc2 = pc2.reshape(NC, N, 16)

    out = pl.pallas_call(
        _tc2_body,
        grid=grid,
        in_specs=[
            _row_spec(), _part_spec(D), _part_spec(16), _row_spec(), _row_spec(),
            _w_spec(), _b_spec(), _w_spec(),
            _w_spec(), _b_spec(), _w_spec(), _b_spec(), _w_spec(), _b_spec(),
        ],
        out_specs=_row_spec(),
        out_shape=jax.ShapeDtypeStruct((N, D), jnp.float32),
    )(h1, pf2, pc2, res1, res2,
      conv2_Wl, conv2_bl.reshape(1, D), conv2_Wr,
      fc1_W, fc1_b.reshape(1, D), fc2_W, fc2_b.reshape(1, D),
      fc3_W, fc3_b.reshape(1, D))

    return out


# speedup vs baseline: 4.7549x; 4.7549x over previous
import functools

import jax
import jax.numpy as jnp
from jax import lax
from jax.experimental import pallas as pl
from jax.experimental.pallas import tpu as pltpu
from jax.experimental.pallas import tpu_sc as plsc

N = 10000
E = 320000
D = 128

NC = 2
NS = 16
NW = NC * NS
EPW = E // NW
CHUNK = 80
NCHUNK = EPW // CHUNK
NP = 10240
RPS = NP // NS


def _sc_aggregate_call(x, src, dst, zf, zc, ones_c):
    mesh = plsc.VectorSubcoreMesh(core_axis_name="c", subcore_axis_name="s")

    @functools.partial(
        pl.kernel,
        out_type=[
            jax.ShapeDtypeStruct((NC * NP, D), jnp.float32),
            jax.ShapeDtypeStruct((NC * NP, 16), jnp.float32),
        ],
        mesh=mesh,
        compiler_params=pltpu.CompilerParams(use_tc_tiling_on_sc=False),
        scratch_types=[
            pltpu.VMEM((CHUNK,), jnp.int32),
            pltpu.VMEM((CHUNK,), jnp.int32),
            pltpu.VMEM((CHUNK, D), jnp.float32),
            pltpu.VMEM((CHUNK, 16), jnp.float32),
            pltpu.VMEM((CHUNK, 16), jnp.float32),
            pltpu.VMEM_SHARED((NP, D), jnp.float32),
            pltpu.VMEM_SHARED((NP, 16), jnp.float32),
            pltpu.SemaphoreType.DMA,
        ],
    )
    def k(x_hbm, src_hbm, dst_hbm, zf_hbm, zc_hbm, ones_hbm,
          outf_hbm, outc_hbm, srcv, dstv, rows, onesv, cstage,
          acc, cacc, sem):
        c = lax.axis_index("c")
        s = lax.axis_index("s")
        wid = s * NC + c
        nstripe = RPS // CHUNK

        pltpu.sync_copy(zf_hbm.at[pl.ds(0, CHUNK)], rows)
        pltpu.sync_copy(zc_hbm.at[pl.ds(0, CHUNK)], cstage)
        for j in range(nstripe):
            pltpu.sync_copy(rows, acc.at[pl.ds(s * RPS + j * CHUNK, CHUNK)])
            pltpu.sync_copy(cstage, cacc.at[pl.ds(s * RPS + j * CHUNK, CHUNK)])
        pltpu.sync_copy(ones_hbm, onesv)
        plsc.subcore_barrier()

        base = wid * EPW

        def body(i, carry):
            b = base + i * CHUNK
            pltpu.sync_copy(src_hbm.at[pl.ds(b, CHUNK)], srcv)
            pltpu.sync_copy(dst_hbm.at[pl.ds(b, CHUNK)], dstv)
            pltpu.async_copy(x_hbm.at[srcv], rows, sem).wait()
            pltpu.sync_copy(rows, acc.at[dstv], add=True)
            pltpu.sync_copy(onesv, cacc.at[dstv], add=True)
            return carry

        lax.fori_loop(0, NCHUNK, body, 0)
        plsc.subcore_barrier()

        ob = c * NP + s * RPS
        for j in range(nstripe):
            pltpu.sync_copy(acc.at[pl.ds(s * RPS + j * CHUNK, CHUNK)], rows)
            pltpu.sync_copy(rows, outf_hbm.at[pl.ds(ob + j * CHUNK, CHUNK)])
            pltpu.sync_copy(cacc.at[pl.ds(s * RPS + j * CHUNK, CHUNK)], cstage)
            pltpu.sync_copy(cstage, outc_hbm.at[pl.ds(ob + j * CHUNK, CHUNK)])

    return k(x, src, dst, zf, zc, ones_c)


def _elu(v):
    return jnp.where(v > 0, v, jnp.exp(jnp.minimum(v, 0.0)) - 1.0)


def _softplus(v):
    return jnp.maximum(v, 0.0) + jnp.log(1.0 + jnp.exp(-jnp.abs(v)))


def _dot_t(a, w):
    return lax.dot_general(a, w, (((1,), (1,)), ((), ())),
                           preferred_element_type=jnp.float32)


TCB = 1000


def _tc1_body(x, pf, pc, wl1, bl1, wr1, r1w, r1b, r2w, r2b,
              h1_o, res1_o, res2_o):
    agg = pf[0] + pf[1]
    cnt = (pc[0] + pc[1])[:, 0:1]
    mean = agg / jnp.maximum(cnt, 1.0)
    lin = _dot_t(mean, wl1[...]) + bl1[...] + _dot_t(x[...], wr1[...])
    h1_o[...] = _elu(lin)
    res1_o[...] = _dot_t(x[...], r1w[...]) + r1b[...]
    res2_o[...] = _dot_t(x[...], r2w[...]) + r2b[...]


def _tc2_body(h1, pf, pc, res1, res2, wl2, bl2, wr2,
              f1w, f1b, f2w, f2b, f3w, f3b, out_o):
    agg = pf[0] + pf[1]
    cnt = (pc[0] + pc[1])[:, 0:1]
    mean = agg / jnp.maximum(cnt, 1.0)
    h = _elu(_dot_t(mean, wl2[...]) + bl2[...] + _dot_t(h1[...], wr2[...]))
    h = h + res1[...]
    h = _elu(_dot_t(h, f1w[...]) + f1b[...])
    h = _elu(_dot_t(h, f2w[...]) + f2b[...]) + res2[...]
    out_o[...] = _softplus(_dot_t(h, f3w[...]) + f3b[...])


def _row_spec(width=D):
    return pl.BlockSpec((TCB, width), lambda i: (i, 0))


def _part_spec(width):
    return pl.BlockSpec((NC, TCB, width), lambda i: (0, i, 0))


def _w_spec():
    return pl.BlockSpec((D, D), lambda i: (0, 0))


def _b_spec():
    return pl.BlockSpec((1, D), lambda i: (0, 0))


def kernel(x, edge_index, conv1_Wl, conv1_bl, conv1_Wr, conv2_Wl, conv2_bl,
           conv2_Wr, res1_W, res1_b, res2_W, res2_b, fc1_W, fc1_b, fc2_W,
           fc2_b, fc3_W, fc3_b):
    src = edge_index[0].astype(jnp.int32)
    dst = edge_index[1].astype(jnp.int32)
    zf = jnp.zeros((NP, D), jnp.float32)
    zc = jnp.zeros((NP, 16), jnp.float32)
    ones_c = jnp.ones((CHUNK, 16), jnp.float32)

    pf1, pc1 = _sc_aggregate_call(x, src, dst, zf, zc, ones_c)
    pf1 = pf1.reshape(NC, NP, D)
    pc1 = pc1.reshape(NC, NP, 16)

    grid = (N // TCB,)
    h1, res1, res2 = pl.pallas_call(
        _tc1_body,
        grid=grid,
        in_specs=[
            _row_spec(), _part_spec(D), _part_spec(16),
            _w_spec(), _b_spec(), _w_spec(),
            _w_spec(), _b_spec(), _w_spec(), _b_spec(),
        ],
        out_specs=[_row_spec(), _row_spec(), _row_spec()],
        out_shape=[jax.ShapeDtypeStruct((N, D), jnp.float32)] * 3,
    )(x, pf1, pc1, conv1_Wl, conv1_bl.reshape(1, D), conv1_Wr,
      res1_W, res1_b.reshape(1, D), res2_W, res2_b.reshape(1, D))

    pf2, pc2 = _sc_aggregate_call(h1, src, dst, zf, zc, ones_c)
    pf2 = pf2.reshape(NC, NP, D)
    pc2 = pc2.reshape(NC, NP, 16)

    out = pl.pallas_call(
        _tc2_body,
        grid=grid,
        in_specs=[
            _row_spec(), _part_spec(D), _part_spec(16), _row_spec(), _row_spec(),
            _w_spec(), _b_spec(), _w_spec(),
            _w_spec(), _b_spec(), _w_spec(), _b_spec(), _w_spec(), _b_spec(),
        ],
        out_specs=_row_spec(),
        out_shape=jax.ShapeDtypeStruct((N, D), jnp.float32),
    )(h1, pf2, pc2, res1, res2,
      conv2_Wl, conv2_bl.reshape(1, D), conv2_Wr,
      fc1_W, fc1_b.reshape(1, D), fc2_W, fc2_b.reshape(1, D),
      fc3_W, fc3_b.reshape(1, D))

    return out
